# Initial kernel scaffold; baseline (speedup 1.0000x reference)
import jax
import jax.numpy as jnp
from jax.experimental import pallas as pl


def kernel(x, adj, W1, b1, W2, b2):
    raise NotImplementedError("write your pallas kernel here")


# speedup vs baseline: 57.5473x; 57.5473x over previous
import functools

import jax
import jax.numpy as jnp
from jax.experimental import pallas as pl
from jax.experimental.pallas import tpu as pltpu


_VMEM_LIMIT = pltpu.CompilerParams(vmem_limit_bytes=100 * 1024 * 1024)


def _pick_blk(n):
    for blk in (200, 100, 50, 25, 8):
        if n % blk == 0:
            return blk
    return n


def _deg_kernel(adj_ref, dinv_ref, *, nblk, blk):
    j = pl.program_id(0)

    @pl.when(j == 0)
    def _():
        dinv_ref[...] = jnp.ones_like(dinv_ref)

    ones = jnp.ones((blk, 1), dtype=jnp.float32)
    dinv_ref[...] += jax.lax.dot_general(
        adj_ref[...], ones, (((0,), (0,)), ((), ())),
        preferred_element_type=jnp.float32)

    @pl.when(j == nblk - 1)
    def _():
        d = dinv_ref[...]
        dinv_ref[...] = jnp.where(d > 0, jax.lax.rsqrt(d), 0.0)


def _gcn_kernel(adj_ref, x_ref, w_ref, b_ref, dinv_ref, out_ref, u_ref,
                *, nblk, blk, relu, logsm, temp):
    j = pl.program_id(0)

    @pl.when(j == 0)
    def _():
        u_ref[...] = dinv_ref[...] * jnp.dot(
            x_ref[...], w_ref[...], preferred_element_type=jnp.float32)
        out_ref[...] = jnp.zeros_like(out_ref)

    u_blk = u_ref[pl.ds(j * blk, blk), :]
    out_ref[...] += jax.lax.dot_general(
        adj_ref[...], u_blk, (((0,), (0,)), ((), ())),
        preferred_element_type=jnp.float32)

    @pl.when(j == nblk - 1)
    def _():
        v = dinv_ref[...] * (out_ref[...] + u_ref[...]) + b_ref[...]
        if relu:
            v = jnp.maximum(v, 0.0)
        if logsm:
            t = v * (1.0 / temp)
            m = jnp.max(t, axis=1, keepdims=True)
            s = t - m
            v = s - jnp.log(jnp.sum(jnp.exp(s), axis=1, keepdims=True))
        out_ref[...] = v


def kernel(x, adj, W1, b1, W2, b2):
    n = adj.shape[0]
    blk = _pick_blk(n)
    nblk = n // blk

    dinv = pl.pallas_call(
        functools.partial(_deg_kernel, nblk=nblk, blk=blk),
        grid=(nblk,),
        in_specs=[pl.BlockSpec((blk, n), lambda j: (j, 0))],
        out_specs=pl.BlockSpec((n, 1), lambda j: (0, 0)),
        out_shape=jax.ShapeDtypeStruct((n, 1), jnp.float32),
        compiler_params=_VMEM_LIMIT,
    )(adj)

    def layer(h, w, b, relu, logsm, temp):
        f = w.shape[1]
        return pl.pallas_call(
            functools.partial(_gcn_kernel, nblk=nblk, blk=blk, relu=relu,
                              logsm=logsm, temp=temp),
            grid=(nblk,),
            in_specs=[
                pl.BlockSpec((blk, n), lambda j: (j, 0)),
                pl.BlockSpec((n, h.shape[1]), lambda j: (0, 0)),
                pl.BlockSpec(w.shape, lambda j: (0, 0)),
                pl.BlockSpec((1, f), lambda j: (0, 0)),
                pl.BlockSpec((n, 1), lambda j: (0, 0)),
            ],
            out_specs=pl.BlockSpec((n, f), lambda j: (0, 0)),
            out_shape=jax.ShapeDtypeStruct((n, f), jnp.float32),
            scratch_shapes=[pltpu.VMEM((n, f), jnp.float32)],
            compiler_params=_VMEM_LIMIT,
        )(adj, h, w, b.reshape(1, f), dinv)

    h1 = layer(x, W1, b1, relu=True, logsm=False, temp=1.0)
    out = layer(h1, W2, b2, relu=False, logsm=True, temp=0.2)
    return out
